# Initial kernel scaffold; baseline (speedup 1.0000x reference)
import jax
import jax.numpy as jnp
from jax.experimental import pallas as pl


def kernel(x, emb, W1, b1, W2, b2):
    raise NotImplementedError("write your pallas kernel here")


# speedup vs baseline: 57.9328x; 57.9328x over previous
import functools

import jax
import jax.numpy as jnp
from jax import lax
from jax.experimental import pallas as pl
from jax.experimental.pallas import tpu as pltpu
from jax.experimental.pallas import tpu_sc as plsc

_VOCAB = 100000
_E = 64
_H = 256
_O = 5
_B = 16384
_L = 200
_NC = 2
_NS = 16
_NW = _NC * _NS
_VP = 102400
_COLS = -(-_L // _NW)
_KB = 12800


def _hist_body(xT, counts, idx_v, cnt_v):
    wid = lax.axis_index("s") * _NC + lax.axis_index("c")

    def _zero(i, c):
        cnt_v[pl.ds(i * 16, 16)] = jnp.zeros((16,), jnp.float32)
        return c
    lax.fori_loop(0, _VP // 16, _zero, 0, unroll=8)

    ones = jnp.ones((16,), jnp.float32)
    zeros = jnp.zeros((16,), jnp.float32)

    def _col(j, c):
        l = wid + j * _NW

        @pl.when(l < _L)
        def _():
            pltpu.sync_copy(xT.at[l], idx_v)

            def _scat(i, c2):
                v = idx_v[pl.ds(i * 16, 16)]
                plsc.addupdate_scatter(cnt_v, [v], ones)
                return c2
            lax.fori_loop(0, _B // 16, _scat, 0, unroll=8)

            pltpu.sync_copy(cnt_v, counts.at[l])

            def _unscat(i, c2):
                v = idx_v[pl.ds(i * 16, 16)]
                plsc.store_scatter(cnt_v, [v], zeros)
                return c2
            lax.fori_loop(0, _B // 16, _unscat, 0, unroll=8)
        return c

    lax.fori_loop(0, _COLS, _col, 0)


_hist = functools.partial(
    pl.kernel,
    out_type=jax.ShapeDtypeStruct((_L, _VP), jnp.float32),
    mesh=plsc.VectorSubcoreMesh(core_axis_name="c", subcore_axis_name="s"),
    scratch_types=[
        pltpu.VMEM((_B,), jnp.int32),
        pltpu.VMEM((_VP,), jnp.float32),
    ],
    compiler_params=pltpu.CompilerParams(needs_layout_passes=False),
)(_hist_body)


def _mlp_body(counts_ref, emb_ref, w1t_ref, b1_ref, w2t_ref, b2_ref,
              out_ref, acc_ref):
    k = pl.program_id(0)

    @pl.when(k == 0)
    def _():
        acc_ref[...] = jnp.zeros_like(acc_ref)

    acc_ref[...] += jnp.dot(counts_ref[...], emb_ref[...],
                            preferred_element_type=jnp.float32)

    @pl.when(k == pl.num_programs(0) - 1)
    def _():
        s = acc_ref[...] * (1.0 / _B)
        h1 = jnp.tanh(s @ w1t_ref[...] + b1_ref[...])
        h2 = h1 @ w2t_ref[...] + b2_ref[...]
        m = jnp.max(h2, axis=0, keepdims=True)
        lse = jnp.log(jnp.sum(jnp.exp(h2 - m), axis=0, keepdims=True)) + m
        out_ref[...] = h2 - lse


_mlp = pl.pallas_call(
    _mlp_body,
    grid=(_VP // _KB,),
    in_specs=[
        pl.BlockSpec((_L, _KB), lambda k: (0, k)),
        pl.BlockSpec((_KB, _E), lambda k: (k, 0)),
        pl.BlockSpec((_E, _H), lambda k: (0, 0)),
        pl.BlockSpec((1, _H), lambda k: (0, 0)),
        pl.BlockSpec((_H, _O), lambda k: (0, 0)),
        pl.BlockSpec((1, _O), lambda k: (0, 0)),
    ],
    out_specs=pl.BlockSpec((_L, _O), lambda k: (0, 0)),
    out_shape=jax.ShapeDtypeStruct((_L, _O), jnp.float32),
    scratch_shapes=[pltpu.VMEM((_L, _E), jnp.float32)],
)


def kernel(x, emb, W1, b1, W2, b2):
    xT = x.T.astype(jnp.int32)
    counts = _hist(xT)
    emb_p = jnp.zeros((_VP, _E), jnp.float32).at[:_VOCAB].set(emb)
    return _mlp(counts, emb_p, W1.T, b1.reshape(1, _H),
                W2.T, b2.reshape(1, _O))


# speedup vs baseline: 78.2867x; 1.3513x over previous
import functools

import jax
import jax.numpy as jnp
from jax import lax
from jax.experimental import pallas as pl
from jax.experimental.pallas import tpu as pltpu
from jax.experimental.pallas import tpu_sc as plsc

_VOCAB = 100000
_E = 64
_H = 256
_O = 5
_B = 16384
_L = 200
_NC = 2
_NS = 16
_NW = _NC * _NS
_VP = 102400
_COLS = -(-_L // _NW)
_KB = 12800


def _hist_body(xT, counts, idx_v, cnt_v):
    wid = lax.axis_index("s") * _NC + lax.axis_index("c")

    @plsc.parallel_loop(0, _VP, 16, unroll=8)
    def _zero(i):
        cnt_v[pl.ds(i, 16)] = jnp.zeros((16,), jnp.float32)

    ones = jnp.ones((16,), jnp.float32)
    zeros = jnp.zeros((16,), jnp.float32)

    def _col(j, c):
        l = wid + j * _NW

        @pl.when(l < _L)
        def _():
            pltpu.sync_copy(xT.at[l], idx_v)

            @plsc.parallel_loop(0, _B, 16, unroll=8)
            def _scat(i):
                v = idx_v[pl.ds(i, 16)]
                plsc.addupdate_scatter(cnt_v, [v], ones)

            pltpu.sync_copy(cnt_v, counts.at[l])

            @plsc.parallel_loop(0, _B, 16, unroll=8)
            def _unscat(i):
                v = idx_v[pl.ds(i, 16)]
                plsc.store_scatter(cnt_v, [v], zeros)
        return c

    lax.fori_loop(0, _COLS, _col, 0)


_hist = functools.partial(
    pl.kernel,
    out_type=jax.ShapeDtypeStruct((_L, _VP), jnp.float32),
    mesh=plsc.VectorSubcoreMesh(core_axis_name="c", subcore_axis_name="s"),
    scratch_types=[
        pltpu.VMEM((_B,), jnp.int32),
        pltpu.VMEM((_VP,), jnp.float32),
    ],
    compiler_params=pltpu.CompilerParams(needs_layout_passes=False),
)(_hist_body)


def _mlp_body(counts_ref, emb_ref, w1t_ref, b1_ref, w2t_ref, b2_ref,
              out_ref, acc_ref):
    k = pl.program_id(0)

    @pl.when(k == 0)
    def _():
        acc_ref[...] = jnp.zeros_like(acc_ref)

    acc_ref[...] += jnp.dot(counts_ref[...], emb_ref[...],
                            preferred_element_type=jnp.float32)

    @pl.when(k == pl.num_programs(0) - 1)
    def _():
        s = acc_ref[...] * (1.0 / _B)
        h1 = jnp.tanh(s @ w1t_ref[...] + b1_ref[...])
        h2 = h1 @ w2t_ref[...] + b2_ref[...]
        m = jnp.max(h2, axis=0, keepdims=True)
        lse = jnp.log(jnp.sum(jnp.exp(h2 - m), axis=0, keepdims=True)) + m
        out_ref[...] = h2 - lse


_mlp = pl.pallas_call(
    _mlp_body,
    grid=(_VP // _KB,),
    in_specs=[
        pl.BlockSpec((_L, _KB), lambda k: (0, k)),
        pl.BlockSpec((_KB, _E), lambda k: (k, 0)),
        pl.BlockSpec((_E, _H), lambda k: (0, 0)),
        pl.BlockSpec((1, _H), lambda k: (0, 0)),
        pl.BlockSpec((_H, _O), lambda k: (0, 0)),
        pl.BlockSpec((1, _O), lambda k: (0, 0)),
    ],
    out_specs=pl.BlockSpec((_L, _O), lambda k: (0, 0)),
    out_shape=jax.ShapeDtypeStruct((_L, _O), jnp.float32),
    scratch_shapes=[pltpu.VMEM((_L, _E), jnp.float32)],
)


def kernel(x, emb, W1, b1, W2, b2):
    xT = x.T.astype(jnp.int32)
    counts = _hist(xT)
    emb_p = jnp.zeros((_VP, _E), jnp.float32).at[:_VOCAB].set(emb)
    return _mlp(counts, emb_p, W1.T, b1.reshape(1, _H),
                W2.T, b2.reshape(1, _O))
